# baseline (speedup 1.0000x reference)
import functools

import jax
import jax.numpy as jnp
from jax import lax
from jax.experimental import pallas as pl
from jax.experimental.pallas import tpu as pltpu
from jax.experimental.pallas import tpu_sc as plsc

_SUB = 72
_NSUB = 8
_SBK = _SUB * _NSUB


def _pre_body(p1_ref, w0_ref, b0_ref, w1_ref, b1_ref, h_ref):
    t = jnp.dot(p1_ref[...], w0_ref[...], preferred_element_type=jnp.float32)
    t = jnp.tanh(t + b0_ref[...])
    t = jnp.dot(t, w1_ref[...], preferred_element_type=jnp.float32)
    h_ref[...] = jnp.tanh(t + b1_ref[...])


def _pre_ff(p1, w0, b0, w1, b1):
    n, d = p1.shape
    return pl.pallas_call(
        _pre_body,
        out_shape=jax.ShapeDtypeStruct((n, d), jnp.float32),
    )(p1, w0, b0.reshape(1, d), w1, b1.reshape(1, d))


def _make_gather(n_half, d):
    info = plsc.get_sparse_core_info()
    nc, ns = info.num_cores, info.num_subcores
    nw = nc * ns
    sub, grp = _SUB, _NSUB
    blk = sub * grp
    ngrp_total = (2 * n_half) // blk
    ngrp_half = n_half // blk
    assert n_half % blk == 0
    base_cnt = ngrp_total // nw
    extra = ngrp_total - base_cnt * nw

    def body(h_hbm, idx_hbm, out_hbm, idxb, rows, sem):
        c = lax.axis_index("c")
        s = lax.axis_index("s")
        wid = s * nc + c
        ngrp = base_cnt + jnp.where(wid < extra, 1, 0)

        def group(k, carry):
            g = wid + nw * k
            off = pl.multiple_of(g * blk, 8)
            half = jnp.where(g < ngrp_half, 0, 1)
            hoff = pl.multiple_of((g - half * ngrp_half) * blk, 8)
            pltpu.sync_copy(idx_hbm.at[pl.ds(off, blk)], idxb)
            cps = [
                pltpu.async_copy(
                    h_hbm.at[idxb.at[pl.ds(k2 * sub, sub)]],
                    rows.at[pl.ds(k2 * sub, sub)],
                    sem,
                )
                for k2 in range(grp)
            ]
            for cp in cps:
                cp.wait()
            pltpu.sync_copy(rows, out_hbm.at[half, pl.ds(hoff, blk)])
            return carry

        lax.fori_loop(0, ngrp, group, 0)

    mesh = plsc.VectorSubcoreMesh(core_axis_name="c", subcore_axis_name="s")
    return pl.kernel(
        body,
        out_type=jax.ShapeDtypeStruct((2, n_half, d), jnp.float32),
        mesh=mesh,
        scratch_types=[
            pltpu.VMEM((blk,), jnp.int32),
            pltpu.VMEM((blk, d), jnp.float32),
            pltpu.SemaphoreType.DMA,
        ],
    )


def _pair_body(nb, d, hi_ref, hj_ref, basis_ref, w2_ref, b2_ref,
               ii0_ref, ii1_ref, eqw_ref, eqb_ref, i1_ref, i1g_ref):
    cat = jnp.concatenate([hi_ref[0], hj_ref[0]], axis=1)
    inter = jnp.dot(cat.astype(jnp.bfloat16), w2_ref[...],
                    preferred_element_type=jnp.float32)
    inter = jnp.tanh(inter + b2_ref[...])
    acc = inter[:, 0:d] * basis_ref[:, 0:1]
    for b in range(1, nb):
        acc = acc + inter[:, b * d:(b + 1) * d] * basis_ref[:, b:b + 1]
    i1 = jnp.tanh(jnp.dot(acc, ii0_ref[...], preferred_element_type=jnp.float32))
    i1 = jnp.tanh(jnp.dot(i1, ii1_ref[...], preferred_element_type=jnp.float32))
    i1_ref[...] = i1
    g = jnp.dot(i1, eqw_ref[...], preferred_element_type=jnp.float32)
    i1g_ref[...] = jnp.tanh(g + eqb_ref[...])


def _pair_ff(hc, basis, w2, b2, ii0, ii1, eqw, eqb, n_pad, d, nb, bp):
    nblk = n_pad // bp
    assert n_pad % bp == 0
    full = lambda *shape: pl.BlockSpec(shape, lambda m: (0,) * len(shape))
    return pl.pallas_call(
        functools.partial(_pair_body, nb, d),
        grid=(nblk,),
        in_specs=[
            pl.BlockSpec((1, bp, d), lambda m: (0, m, 0)),
            pl.BlockSpec((1, bp, d), lambda m: (1, m, 0)),
            pl.BlockSpec((bp, nb), lambda m: (m, 0)),
            full(2 * d, d * nb),
            full(1, d * nb),
            full(d, d),
            full(d, d),
            full(d, d),
            full(1, d),
        ],
        out_specs=[
            pl.BlockSpec((bp, d), lambda m: (m, 0)),
            pl.BlockSpec((bp, d), lambda m: (m, 0)),
        ],
        out_shape=[
            jax.ShapeDtypeStruct((n_pad, d), jnp.float32),
            jax.ShapeDtypeStruct((n_pad, d), jnp.float32),
        ],
    )(hc, hc, basis, w2, b2, ii0, ii1, eqw, eqb)


def _make_scatter(n_atoms, n_pad, d):
    info = plsc.get_sparse_core_info()
    nc, ns = info.num_cores, info.num_subcores
    nw = nc * ns
    sub, nsub, sbk = _SUB, _NSUB, _SBK
    nsbk_total = n_pad // sbk
    assert n_pad % sbk == 0
    base_cnt = nsbk_total // nw
    extra = nsbk_total - base_cnt * nw
    slab_rows = ((n_atoms + ns - 1) // ns + 7) // 8 * 8
    nacc = ns * slab_rows
    nv = d // 16

    def body(i1_hbm, i1g_hbm, p30, p31, p32, d30, d31, d32,
             i2d_hbm, j2d_hbm, zeros_hbm, s_hbm,
             acc, ib2, jb2, db, rows, g1, gsem, lsem, ssem):
        c = lax.axis_index("c")
        s = lax.axis_index("s")
        w = s * nc + c
        nsbk = base_cnt + jnp.where(w < extra, 1, 0)
        slab = s * slab_rows

        def zero_acc():
            pltpu.sync_copy(zeros_hbm, acc.at[pl.ds(slab, slab_rows)])

        def dump(phase):
            pltpu.sync_copy(acc.at[pl.ds(slab, slab_rows)],
                            s_hbm.at[phase, c, pl.ds(slab, slab_rows)])

        def run_phase(rows_hbm, px, dx):
            eq = px is not None

            def sbk_body(k, carry):
                sb = w + nw * k
                rb = pl.multiple_of(sb * nsub, 8)
                pltpu.sync_copy(i2d_hbm.at[pl.ds(rb, nsub)], ib2)
                if eq:
                    pltpu.sync_copy(j2d_hbm.at[pl.ds(rb, nsub)], jb2)

                def fire(r):
                    off = pl.multiple_of(sb * sbk + r * sub, 8)
                    pltpu.async_copy(rows_hbm.at[pl.ds(off, sub)],
                                     g1.at[r % 3], lsem)
                    if eq:
                        pltpu.async_copy(dx.at[pl.ds(off, sub)],
                                         db.at[r % 2, pl.ds(0, sub)], lsem)
                        pltpu.async_copy(px.at[jb2.at[r]], rows.at[r % 2],
                                         gsem)

                def wait_loads():
                    pltpu.make_async_copy(i1_hbm.at[pl.ds(0, sub)],
                                          g1.at[0], lsem).wait()
                    if eq:
                        pltpu.make_async_copy(
                            d30.at[pl.ds(0, sub)],
                            db.at[0, pl.ds(0, sub)], lsem).wait()
                        pltpu.make_async_copy(p30.at[pl.ds(0, sub)],
                                              rows.at[0], gsem).wait()

                def wait_scatter():
                    pltpu.make_async_copy(g1.at[0], acc.at[ib2.at[0]],
                                          ssem).wait()

                fire(0)
                for r in range(nsub):
                    l3, l2 = r % 3, r % 2
                    wait_loads()
                    if r + 1 < nsub:
                        if r >= 2:
                            wait_scatter()
                        fire(r + 1)
                    if eq:
                        @plsc.parallel_loop(0, sub, unroll=4)
                        def _pair(k2):
                            dvec = jnp.full((16,),
                                            db[l2, pl.ds(k2, 16)][0],
                                            jnp.float32)
                            for v in range(nv):
                                vs = pl.ds(v * 16, 16)
                                g1[l3, k2, vs] = ((rows[l2, k2, vs] + dvec)
                                                  * g1[l3, k2, vs])
                    pltpu.async_copy(g1.at[l3], acc.at[ib2.at[r]], ssem,
                                     add=True)
                for _ in range(3):
                    wait_scatter()
                return carry

            lax.fori_loop(0, nsbk, sbk_body, 0)

        zero_acc()
        plsc.subcore_barrier()

        run_phase(i1_hbm, None, None)
        plsc.subcore_barrier()
        dump(0)
        plsc.subcore_barrier()

        for x, (px, dx) in enumerate(((p30, d30), (p31, d31), (p32, d32))):
            zero_acc()
            plsc.subcore_barrier()
            run_phase(i1g_hbm, px, dx)
            plsc.subcore_barrier()
            dump(1 + x)
            plsc.subcore_barrier()

    mesh = plsc.VectorSubcoreMesh(core_axis_name="c", subcore_axis_name="s")
    return pl.kernel(
        body,
        out_type=jax.ShapeDtypeStruct((4, nc, nacc, d), jnp.float32),
        mesh=mesh,
        scratch_types=[
            pltpu.VMEM_SHARED((nacc, d), jnp.float32),
            pltpu.VMEM((nsub, sub), jnp.int32),
            pltpu.VMEM((nsub, sub), jnp.int32),
            pltpu.VMEM((2, sub + 16), jnp.float32),
            pltpu.VMEM((2, sub, d), jnp.float32),
            pltpu.VMEM((3, sub, d), jnp.float32),
            pltpu.SemaphoreType.DMA,
            pltpu.SemaphoreType.DMA,
            pltpu.SemaphoreType.DMA,
        ],
    ), nacc, slab_rows


def _fin_body(nch, *refs):
    s_refs = refs[:nch]
    pw0_ref, pw1_ref, ew0_ref, ew1_ref, o1_ref, o3_ref = refs[nch:]
    t = sum(sr[0, h] for sr in s_refs for h in range(2))
    t = jnp.dot(t, pw0_ref[...], preferred_element_type=jnp.float32)
    o1_ref[...] = jnp.dot(t, pw1_ref[...], preferred_element_type=jnp.float32)
    for x in range(3):
        u = sum(sr[1 + x, h] for sr in s_refs for h in range(2))
        u = jnp.dot(u, ew0_ref[...], preferred_element_type=jnp.float32)
        u = jnp.dot(u, ew1_ref[...], preferred_element_type=jnp.float32)
        o3_ref[:, x, :] = u


def _finalize(s_list, pw0, pw1, ew0, ew1, n_atoms, d, ab):
    full = lambda *shape: pl.BlockSpec(shape, lambda m: (0,) * len(shape))
    nch = len(s_list)
    return pl.pallas_call(
        functools.partial(_fin_body, nch),
        grid=(n_atoms // ab,),
        in_specs=[
            pl.BlockSpec((4, 2, ab, d), lambda m: (0, 0, m, 0))
            for _ in range(nch)
        ] + [full(d, d), full(d, d), full(d, d), full(d, d)],
        out_specs=[
            pl.BlockSpec((ab, d), lambda m: (m, 0)),
            pl.BlockSpec((ab, 3, d), lambda m: (m, 0, 0)),
        ],
        out_shape=[
            jax.ShapeDtypeStruct((n_atoms, d), jnp.float32),
            jax.ShapeDtypeStruct((n_atoms, 3, d), jnp.float32),
        ],
    )(*s_list, pw0, pw1, ew0, ew1)


def kernel(p1, p3, d3, basis, ind_2, pp_pre_W0, pp_pre_b0, pp_pre_W1,
           pp_pre_b1, pi_W0, pi_b0, ii_W0, ii_W1, pp_post_W0, pp_post_W1,
           eq_ii_W0, eq_ii_b0, eq_pp_W0, eq_pp_W1):
    n_atoms, d = p1.shape
    n_pairs = ind_2.shape[0]
    nb = basis.shape[1]
    nch = 3
    unit = _SBK * nch
    n_pad = (n_pairs + unit - 1) // unit * unit
    hp = n_pad // nch
    pad = n_pad - n_pairs

    i_idx = ind_2[:, 0]
    j_idx = ind_2[:, 1]
    pad_i = jnp.full((pad,), n_atoms + 8, jnp.int32)
    i_pad = jnp.concatenate([i_idx, pad_i])
    j_pad = jnp.concatenate([j_idx, jnp.zeros((pad,), jnp.int32)])
    d3_pad = jnp.concatenate([d3, jnp.zeros((pad, 3), d3.dtype)])
    basis_pad = jnp.concatenate([basis, jnp.zeros((pad, nb), basis.dtype)])

    h = _pre_ff(p1, pp_pre_W0, pp_pre_b0, pp_pre_W1, pp_pre_b1)

    w2 = pi_W0.reshape(2 * d, d, nb).transpose(0, 2, 1).reshape(2 * d, d * nb)
    w2 = w2.astype(jnp.bfloat16)
    b2 = pi_b0.reshape(d, nb).T.reshape(1, d * nb)
    eqb = eq_ii_b0.reshape(1, d)

    gather = _make_gather(hp, d)
    scatter, nacc, slab_rows = _make_scatter(n_atoms, hp, d)
    zeros = jnp.zeros((slab_rows, d), jnp.float32)
    p3x = (p3[:, 0, :], p3[:, 1, :], p3[:, 2, :])

    s_list = []
    for k in range(nch):
        lo, hi = k * hp, (k + 1) * hp
        idx_k = jnp.concatenate([i_pad[lo:hi], j_pad[lo:hi]])
        hc = gather(h, idx_k)
        i1, i1g = _pair_ff(hc, basis_pad[lo:hi], w2, b2, ii_W0, ii_W1,
                           eq_ii_W0, eqb, hp, d, nb, bp=_SBK)
        s_list.append(scatter(
            i1, i1g, *p3x,
            d3_pad[lo:hi, 0], d3_pad[lo:hi, 1], d3_pad[lo:hi, 2],
            i_pad[lo:hi].reshape(hp // _SUB, _SUB),
            j_pad[lo:hi].reshape(hp // _SUB, _SUB),
            zeros,
        ))

    p1_new, p3_new = _finalize(s_list, pp_post_W0, pp_post_W1, eq_pp_W0,
                               eq_pp_W1, n_atoms, d, ab=2000)
    return (p1_new, p3_new)


# speedup vs baseline: 1.0224x; 1.0224x over previous
import functools

import jax
import jax.numpy as jnp
from jax import lax
from jax.experimental import pallas as pl
from jax.experimental.pallas import tpu as pltpu
from jax.experimental.pallas import tpu_sc as plsc

_SUB = 72
_NSUB = 8
_SBK = _SUB * _NSUB


def _pre_body(p1_ref, w0_ref, b0_ref, w1_ref, b1_ref, h_ref):
    t = jnp.dot(p1_ref[...], w0_ref[...], preferred_element_type=jnp.float32)
    t = jnp.tanh(t + b0_ref[...])
    t = jnp.dot(t, w1_ref[...], preferred_element_type=jnp.float32)
    h_ref[...] = jnp.tanh(t + b1_ref[...])


def _pre_ff(p1, w0, b0, w1, b1):
    n, d = p1.shape
    return pl.pallas_call(
        _pre_body,
        out_shape=jax.ShapeDtypeStruct((n, d), jnp.float32),
    )(p1, w0, b0.reshape(1, d), w1, b1.reshape(1, d))


def _make_gather(n_half, d):
    info = plsc.get_sparse_core_info()
    nc, ns = info.num_cores, info.num_subcores
    nw = nc * ns
    sub, grp = _SUB, _NSUB
    blk = sub * grp
    ngrp_total = (2 * n_half) // blk
    ngrp_half = n_half // blk
    assert n_half % blk == 0
    base_cnt = ngrp_total // nw
    extra = ngrp_total - base_cnt * nw

    def body(h_hbm, idx_hbm, out_hbm, idxb, rows, sem):
        c = lax.axis_index("c")
        s = lax.axis_index("s")
        wid = s * nc + c
        ngrp = base_cnt + jnp.where(wid < extra, 1, 0)

        def group(k, carry):
            g = wid + nw * k
            off = pl.multiple_of(g * blk, 8)
            half = jnp.where(g < ngrp_half, 0, 1)
            hoff = pl.multiple_of((g - half * ngrp_half) * blk, 8)
            pltpu.sync_copy(idx_hbm.at[pl.ds(off, blk)], idxb)
            cps = [
                pltpu.async_copy(
                    h_hbm.at[idxb.at[pl.ds(k2 * sub, sub)]],
                    rows.at[pl.ds(k2 * sub, sub)],
                    sem,
                )
                for k2 in range(grp)
            ]
            for cp in cps:
                cp.wait()
            pltpu.sync_copy(rows, out_hbm.at[half, pl.ds(hoff, blk)])
            return carry

        lax.fori_loop(0, ngrp, group, 0)

    mesh = plsc.VectorSubcoreMesh(core_axis_name="c", subcore_axis_name="s")
    return pl.kernel(
        body,
        out_type=jax.ShapeDtypeStruct((2, n_half, d), jnp.float32),
        mesh=mesh,
        scratch_types=[
            pltpu.VMEM((blk,), jnp.int32),
            pltpu.VMEM((blk, d), jnp.float32),
            pltpu.SemaphoreType.DMA,
        ],
    )


def _pair_body(nb, d, hi_ref, hj_ref, basis_ref, w2_ref, b2_ref,
               ii0_ref, ii1_ref, eqw_ref, eqb_ref, i1_ref, i1g_ref):
    cat = jnp.concatenate([hi_ref[0], hj_ref[0]], axis=1)
    inter = jnp.dot(cat.astype(jnp.bfloat16), w2_ref[...],
                    preferred_element_type=jnp.float32)
    inter = jnp.tanh(inter + b2_ref[...])
    acc = inter[:, 0:d] * basis_ref[:, 0:1]
    for b in range(1, nb):
        acc = acc + inter[:, b * d:(b + 1) * d] * basis_ref[:, b:b + 1]
    i1 = jnp.tanh(jnp.dot(acc, ii0_ref[...], preferred_element_type=jnp.float32))
    i1 = jnp.tanh(jnp.dot(i1, ii1_ref[...], preferred_element_type=jnp.float32))
    i1_ref[...] = i1
    g = jnp.dot(i1, eqw_ref[...], preferred_element_type=jnp.float32)
    i1g_ref[...] = jnp.tanh(g + eqb_ref[...])


def _pair_ff(hc, basis, w2, b2, ii0, ii1, eqw, eqb, n_pad, d, nb, bp):
    nblk = n_pad // bp
    assert n_pad % bp == 0
    full = lambda *shape: pl.BlockSpec(shape, lambda m: (0,) * len(shape))
    return pl.pallas_call(
        functools.partial(_pair_body, nb, d),
        grid=(nblk,),
        in_specs=[
            pl.BlockSpec((1, bp, d), lambda m: (0, m, 0)),
            pl.BlockSpec((1, bp, d), lambda m: (1, m, 0)),
            pl.BlockSpec((bp, nb), lambda m: (m, 0)),
            full(2 * d, d * nb),
            full(1, d * nb),
            full(d, d),
            full(d, d),
            full(d, d),
            full(1, d),
        ],
        out_specs=[
            pl.BlockSpec((bp, d), lambda m: (m, 0)),
            pl.BlockSpec((bp, d), lambda m: (m, 0)),
        ],
        out_shape=[
            jax.ShapeDtypeStruct((n_pad, d), jnp.float32),
            jax.ShapeDtypeStruct((n_pad, d), jnp.float32),
        ],
    )(hc, hc, basis, w2, b2, ii0, ii1, eqw, eqb)


def _make_scatter(n_atoms, n_pad, d):
    info = plsc.get_sparse_core_info()
    nc, ns = info.num_cores, info.num_subcores
    nw = nc * ns
    sub, nsub, sbk = _SUB, _NSUB, _SBK
    nsbk_total = n_pad // sbk
    assert n_pad % sbk == 0
    base_cnt = nsbk_total // nw
    extra = nsbk_total - base_cnt * nw
    slab_rows = ((n_atoms + ns - 1) // ns + 7) // 8 * 8
    nacc = ns * slab_rows
    nv = d // 16

    def body(i1_hbm, i1g_hbm, p30, p31, p32, d30, d31, d32,
             i2d_hbm, j2d_hbm, zeros_hbm, s_hbm,
             acc, ib2, jb2, db, rows, g1, gsem, lsem, ssem):
        c = lax.axis_index("c")
        s = lax.axis_index("s")
        w = s * nc + c
        nsbk = base_cnt + jnp.where(w < extra, 1, 0)
        slab = s * slab_rows

        def zero_acc():
            pltpu.sync_copy(zeros_hbm, acc.at[pl.ds(slab, slab_rows)])

        def dump(phase):
            pltpu.sync_copy(acc.at[pl.ds(slab, slab_rows)],
                            s_hbm.at[phase, c, pl.ds(slab, slab_rows)])

        def run_phase(rows_hbm, px, dx):
            eq = px is not None

            def sbk_body(k, carry):
                sb = w + nw * k
                rb = pl.multiple_of(sb * nsub, 8)
                pltpu.sync_copy(i2d_hbm.at[pl.ds(rb, nsub)], ib2)
                if eq:
                    pltpu.sync_copy(j2d_hbm.at[pl.ds(rb, nsub)], jb2)

                def fire(r):
                    off = pl.multiple_of(sb * sbk + r * sub, 8)
                    pltpu.async_copy(rows_hbm.at[pl.ds(off, sub)],
                                     g1.at[r % 3], lsem)
                    if eq:
                        pltpu.async_copy(dx.at[pl.ds(off, sub)],
                                         db.at[r % 2, pl.ds(0, sub)], lsem)
                        pltpu.async_copy(px.at[jb2.at[r]], rows.at[r % 2],
                                         gsem)

                def wait_loads():
                    pltpu.make_async_copy(i1_hbm.at[pl.ds(0, sub)],
                                          g1.at[0], lsem).wait()
                    if eq:
                        pltpu.make_async_copy(
                            d30.at[pl.ds(0, sub)],
                            db.at[0, pl.ds(0, sub)], lsem).wait()
                        pltpu.make_async_copy(p30.at[pl.ds(0, sub)],
                                              rows.at[0], gsem).wait()

                def wait_scatter():
                    pltpu.make_async_copy(g1.at[0], acc.at[ib2.at[0]],
                                          ssem).wait()

                fire(0)
                for r in range(nsub):
                    l3, l2 = r % 3, r % 2
                    wait_loads()
                    if r + 1 < nsub:
                        if r >= 2:
                            wait_scatter()
                        fire(r + 1)
                    if eq:
                        @plsc.parallel_loop(0, sub, unroll=4)
                        def _pair(k2):
                            dvec = jnp.full((16,),
                                            db[l2, pl.ds(k2, 16)][0],
                                            jnp.float32)
                            for v in range(nv):
                                vs = pl.ds(v * 16, 16)
                                g1[l3, k2, vs] = ((rows[l2, k2, vs] + dvec)
                                                  * g1[l3, k2, vs])
                    pltpu.async_copy(g1.at[l3], acc.at[ib2.at[r]], ssem,
                                     add=True)
                for _ in range(3):
                    wait_scatter()
                return carry

            lax.fori_loop(0, nsbk, sbk_body, 0)

        zero_acc()
        plsc.subcore_barrier()

        run_phase(i1_hbm, None, None)
        plsc.subcore_barrier()
        dump(0)
        plsc.subcore_barrier()

        for x, (px, dx) in enumerate(((p30, d30), (p31, d31), (p32, d32))):
            zero_acc()
            plsc.subcore_barrier()
            run_phase(i1g_hbm, px, dx)
            plsc.subcore_barrier()
            dump(1 + x)
            plsc.subcore_barrier()

    mesh = plsc.VectorSubcoreMesh(core_axis_name="c", subcore_axis_name="s")
    return pl.kernel(
        body,
        out_type=jax.ShapeDtypeStruct((4, nc, nacc, d), jnp.float32),
        mesh=mesh,
        scratch_types=[
            pltpu.VMEM_SHARED((nacc, d), jnp.float32),
            pltpu.VMEM((nsub, sub), jnp.int32),
            pltpu.VMEM((nsub, sub), jnp.int32),
            pltpu.VMEM((2, sub + 16), jnp.float32),
            pltpu.VMEM((2, sub, d), jnp.float32),
            pltpu.VMEM((3, sub, d), jnp.float32),
            pltpu.SemaphoreType.DMA,
            pltpu.SemaphoreType.DMA,
            pltpu.SemaphoreType.DMA,
        ],
    ), nacc, slab_rows


def _fin_body(nch, *refs):
    s_refs = refs[:nch]
    pw0_ref, pw1_ref, ew0_ref, ew1_ref, o1_ref, o3_ref = refs[nch:]
    t = sum(sr[0, h] for sr in s_refs for h in range(2))
    t = jnp.dot(t, pw0_ref[...], preferred_element_type=jnp.float32)
    o1_ref[...] = jnp.dot(t, pw1_ref[...], preferred_element_type=jnp.float32)
    for x in range(3):
        u = sum(sr[1 + x, h] for sr in s_refs for h in range(2))
        u = jnp.dot(u, ew0_ref[...], preferred_element_type=jnp.float32)
        u = jnp.dot(u, ew1_ref[...], preferred_element_type=jnp.float32)
        o3_ref[:, x, :] = u


def _finalize(s_list, pw0, pw1, ew0, ew1, n_atoms, d, ab):
    full = lambda *shape: pl.BlockSpec(shape, lambda m: (0,) * len(shape))
    nch = len(s_list)
    return pl.pallas_call(
        functools.partial(_fin_body, nch),
        grid=(n_atoms // ab,),
        in_specs=[
            pl.BlockSpec((4, 2, ab, d), lambda m: (0, 0, m, 0))
            for _ in range(nch)
        ] + [full(d, d), full(d, d), full(d, d), full(d, d)],
        out_specs=[
            pl.BlockSpec((ab, d), lambda m: (m, 0)),
            pl.BlockSpec((ab, 3, d), lambda m: (m, 0, 0)),
        ],
        out_shape=[
            jax.ShapeDtypeStruct((n_atoms, d), jnp.float32),
            jax.ShapeDtypeStruct((n_atoms, 3, d), jnp.float32),
        ],
    )(*s_list, pw0, pw1, ew0, ew1)


def kernel(p1, p3, d3, basis, ind_2, pp_pre_W0, pp_pre_b0, pp_pre_W1,
           pp_pre_b1, pi_W0, pi_b0, ii_W0, ii_W1, pp_post_W0, pp_post_W1,
           eq_ii_W0, eq_ii_b0, eq_pp_W0, eq_pp_W1):
    n_atoms, d = p1.shape
    n_pairs = ind_2.shape[0]
    nb = basis.shape[1]
    nch = 2
    unit = _SBK * nch
    n_pad = (n_pairs + unit - 1) // unit * unit
    hp = n_pad // nch
    pad = n_pad - n_pairs

    i_idx = ind_2[:, 0]
    j_idx = ind_2[:, 1]
    pad_i = jnp.full((pad,), n_atoms + 8, jnp.int32)
    i_pad = jnp.concatenate([i_idx, pad_i])
    j_pad = jnp.concatenate([j_idx, jnp.zeros((pad,), jnp.int32)])
    d3_pad = jnp.concatenate([d3, jnp.zeros((pad, 3), d3.dtype)])
    basis_pad = jnp.concatenate([basis, jnp.zeros((pad, nb), basis.dtype)])

    h = _pre_ff(p1, pp_pre_W0, pp_pre_b0, pp_pre_W1, pp_pre_b1)

    w2 = pi_W0.reshape(2 * d, d, nb).transpose(0, 2, 1).reshape(2 * d, d * nb)
    w2 = w2.astype(jnp.bfloat16)
    b2 = pi_b0.reshape(d, nb).T.reshape(1, d * nb)
    eqb = eq_ii_b0.reshape(1, d)

    gather = _make_gather(hp, d)
    scatter, nacc, slab_rows = _make_scatter(n_atoms, hp, d)
    zeros = jnp.zeros((slab_rows, d), jnp.float32)
    p3x = (p3[:, 0, :], p3[:, 1, :], p3[:, 2, :])

    s_list = []
    for k in range(nch):
        lo, hi = k * hp, (k + 1) * hp
        idx_k = jnp.concatenate([i_pad[lo:hi], j_pad[lo:hi]])
        hc = gather(h, idx_k)
        i1, i1g = _pair_ff(hc, basis_pad[lo:hi], w2, b2, ii_W0, ii_W1,
                           eq_ii_W0, eqb, hp, d, nb, bp=_SBK)
        s_list.append(scatter(
            i1, i1g, *p3x,
            d3_pad[lo:hi, 0], d3_pad[lo:hi, 1], d3_pad[lo:hi, 2],
            i_pad[lo:hi].reshape(hp // _SUB, _SUB),
            j_pad[lo:hi].reshape(hp // _SUB, _SUB),
            zeros,
        ))

    p1_new, p3_new = _finalize(s_list, pp_post_W0, pp_post_W1, eq_pp_W0,
                               eq_pp_W1, n_atoms, d, ab=2000)
    return (p1_new, p3_new)
